# baseline (speedup 1.0000x reference)
import jax
import jax.numpy as jnp
from jax.experimental import pallas as pl
from jax.experimental.pallas import tpu as pltpu


def _make_body(NI, NK, BN, BK, NC):
    def _body(f_ref, w1_ref, b1_ref, w2_ref, b2_ref, wh_ref, bh_ref,
              outc_ref, outr_ref, acc_ref):
        k = pl.program_id(0)
        i = pl.program_id(1)
        rows = pl.ds(i * BN, BN)

        part = jnp.dot(f_ref[...], w1_ref[...],
                       preferred_element_type=jnp.float32)
        prev = acc_ref[rows, :]
        acc_new = jnp.where(k > 0, prev + part, part)
        acc_ref[rows, :] = acc_new

        @pl.when(k == NK - 1)
        def _finish():
            x = jnp.maximum(acc_new + b1_ref[...], 0.0)
            x = jnp.dot(x, w2_ref[...], preferred_element_type=jnp.float32)
            x = jnp.maximum(x + b2_ref[...], 0.0)
            y = jnp.dot(x, wh_ref[...], preferred_element_type=jnp.float32)
            y = y + bh_ref[...]
            outc_ref[...] = y[:, :NC]
            outr_ref[...] = y[:, NC:]

    return _body


def kernel(feature_vectors, W1, b1, W2, b2, Wc, bc, Wr, br):
    N, D = feature_vectors.shape
    H = W1.shape[1]
    NC = Wc.shape[1]
    NR = Wr.shape[1]

    BN = 1000
    BK = 896
    assert N % BN == 0 and D % BK == 0
    NI = N // BN
    NK = D // BK
    grid = (NK, NI)

    Wh = jnp.concatenate([Wc, Wr], axis=1)
    bh = jnp.concatenate([bc, br])[None, :]
    b1_2d = b1[None, :]
    b2_2d = b2[None, :]

    outc, outr = pl.pallas_call(
        _make_body(NI, NK, BN, BK, NC),
        grid=grid,
        in_specs=[
            pl.BlockSpec((BN, BK), lambda k, i: (i, k)),
            pl.BlockSpec((BK, H), lambda k, i: (k, 0)),
            pl.BlockSpec((1, H), lambda k, i: (0, 0)),
            pl.BlockSpec((H, H), lambda k, i: (0, 0)),
            pl.BlockSpec((1, H), lambda k, i: (0, 0)),
            pl.BlockSpec((H, NC + NR), lambda k, i: (0, 0)),
            pl.BlockSpec((1, NC + NR), lambda k, i: (0, 0)),
        ],
        out_specs=[
            pl.BlockSpec((BN, NC), lambda k, i: (i, 0)),
            pl.BlockSpec((BN, NR), lambda k, i: (i, 0)),
        ],
        out_shape=[
            jax.ShapeDtypeStruct((N, NC), jnp.float32),
            jax.ShapeDtypeStruct((N, NR), jnp.float32),
        ],
        scratch_shapes=[
            pltpu.VMEM((N, H), jnp.float32),
        ],
        compiler_params=pltpu.CompilerParams(
            dimension_semantics=("arbitrary", "arbitrary"),
        ),
    )(feature_vectors, W1, b1_2d, W2, b2_2d, Wh, bh)
    return outc, outr


# speedup vs baseline: 1.1185x; 1.1185x over previous
import jax
import jax.numpy as jnp
from jax.experimental import pallas as pl
from jax.experimental.pallas import tpu as pltpu


def _make_body(NI, NK, BN, BK, NC):
    def _body(f_ref, w1_ref, b1_ref, w2_ref, b2_ref, wh_ref, bh_ref,
              outc_ref, outr_ref, w1b_ref):
        s = pl.program_id(0)

        @pl.when(s < NK)
        def _warmup():
            w1b_ref[pl.ds(s * BK, BK), :] = w1_ref[...].astype(jnp.bfloat16)

        @pl.when(s >= NK)
        def _main():
            x = jax.lax.dot_general(
                f_ref[...], w1b_ref[...], (((1,), (0,)), ((), ())),
                preferred_element_type=jnp.float32)
            x = jnp.maximum(x + b1_ref[...], 0.0)
            x = jnp.dot(x, w2_ref[...], preferred_element_type=jnp.float32)
            x = jnp.maximum(x + b2_ref[...], 0.0)
            y = jnp.dot(x, wh_ref[...], preferred_element_type=jnp.float32)
            y = y + bh_ref[...]
            outc_ref[...] = y[:, :NC]
            outr_ref[...] = y[:, NC:]

    return _body


def kernel(feature_vectors, W1, b1, W2, b2, Wc, bc, Wr, br):
    N, D = feature_vectors.shape
    H = W1.shape[1]
    NC = Wc.shape[1]
    NR = Wr.shape[1]

    BN = 200
    BK = 448
    assert N % BN == 0 and D % BK == 0
    NI = N // BN
    NK = D // BK
    grid = (NK + NI,)

    Wh = jnp.concatenate([Wc, Wr], axis=1)
    bh = jnp.concatenate([bc, br])[None, :]
    b1_2d = b1[None, :]
    b2_2d = b2[None, :]

    outc, outr = pl.pallas_call(
        _make_body(NI, NK, BN, BK, NC),
        grid=grid,
        in_specs=[
            pl.BlockSpec((BN, D), lambda s: (jnp.clip(s - NK, 0, NI - 1), 0)),
            pl.BlockSpec((BK, H), lambda s: (jnp.minimum(s, NK - 1), 0)),
            pl.BlockSpec((1, H), lambda s: (0, 0)),
            pl.BlockSpec((H, H), lambda s: (0, 0)),
            pl.BlockSpec((1, H), lambda s: (0, 0)),
            pl.BlockSpec((H, NC + NR), lambda s: (0, 0)),
            pl.BlockSpec((1, NC + NR), lambda s: (0, 0)),
        ],
        out_specs=[
            pl.BlockSpec((BN, NC), lambda s: (jnp.clip(s - NK, 0, NI - 1), 0)),
            pl.BlockSpec((BN, NR), lambda s: (jnp.clip(s - NK, 0, NI - 1), 0)),
        ],
        out_shape=[
            jax.ShapeDtypeStruct((N, NC), jnp.float32),
            jax.ShapeDtypeStruct((N, NR), jnp.float32),
        ],
        scratch_shapes=[
            pltpu.VMEM((D, H), jnp.bfloat16),
        ],
        compiler_params=pltpu.CompilerParams(
            dimension_semantics=("arbitrary",),
        ),
    )(feature_vectors, W1, b1_2d, W2, b2_2d, Wh, bh)
    return outc, outr


# speedup vs baseline: 1.1712x; 1.0472x over previous
import jax
import jax.numpy as jnp
from jax.experimental import pallas as pl
from jax.experimental.pallas import tpu as pltpu

_DN = (((1,), (0,)), ((), ()))


def _make_body(NI, NK, BN, BK):
    def _body(f_ref, w1_ref, b1_ref, w2_ref, b2_ref, wh_ref, bh_ref,
              out_ref, acc_ref):
        k = pl.program_id(0)
        i = pl.program_id(1)
        rows = pl.ds(i * BN, BN)

        part = jnp.dot(f_ref[...], w1_ref[...],
                       preferred_element_type=jnp.float32)
        prev = acc_ref[rows, :]
        acc_new = jnp.where(k > 0, prev + part, part)
        acc_ref[rows, :] = acc_new

        @pl.when(k == NK - 1)
        def _finish():
            x = jnp.maximum(acc_new + b1_ref[...], 0.0)
            x = jax.lax.dot_general(x, w2_ref[...], _DN,
                                    preferred_element_type=jnp.float32)
            x = jnp.maximum(x + b2_ref[...], 0.0)
            y = jax.lax.dot_general(x, wh_ref[...], _DN,
                                    preferred_element_type=jnp.float32)
            out_ref[...] = y + bh_ref[...]

    return _body


def kernel(feature_vectors, W1, b1, W2, b2, Wc, bc, Wr, br):
    N, D = feature_vectors.shape
    H = W1.shape[1]
    NC = Wc.shape[1]
    NR = Wr.shape[1]

    BN = 1000
    BK = 1792
    assert N % BN == 0 and D % BK == 0
    NI = N // BN
    NK = D // BK
    grid = (NK, NI)

    Wh = jnp.concatenate([Wc, Wr], axis=1).astype(jnp.bfloat16)
    W2b = W2.astype(jnp.bfloat16)
    bh = jnp.concatenate([bc, br])[None, :]
    b1_2d = b1[None, :]
    b2_2d = b2[None, :]

    out = pl.pallas_call(
        _make_body(NI, NK, BN, BK),
        grid=grid,
        in_specs=[
            pl.BlockSpec((BN, BK), lambda k, i: (i, k)),
            pl.BlockSpec((BK, H), lambda k, i: (k, 0)),
            pl.BlockSpec((1, H), lambda k, i: (0, 0)),
            pl.BlockSpec((H, H), lambda k, i: (0, 0)),
            pl.BlockSpec((1, H), lambda k, i: (0, 0)),
            pl.BlockSpec((H, NC + NR), lambda k, i: (0, 0)),
            pl.BlockSpec((1, NC + NR), lambda k, i: (0, 0)),
        ],
        out_specs=pl.BlockSpec((BN, NC + NR), lambda k, i: (i, 0)),
        out_shape=jax.ShapeDtypeStruct((N, NC + NR), jnp.float32),
        scratch_shapes=[
            pltpu.VMEM((N, H), jnp.float32),
        ],
        compiler_params=pltpu.CompilerParams(
            dimension_semantics=("arbitrary", "arbitrary"),
            vmem_limit_bytes=63 * 1024 * 1024,
        ),
    )(feature_vectors, W1, b1_2d, W2b, b2_2d, Wh, bh)
    return out[:, :NC], out[:, NC:]
